# baseline (speedup 1.0000x reference)
import functools

import jax
import jax.numpy as jnp
from jax import lax
from jax.experimental import pallas as pl
from jax.experimental.pallas import tpu as pltpu
from jax.experimental.pallas import tpu_sc as plsc

N_PTS = 1048576
RES = 256
NC = 2
NS = 16
L = 16
NW = NC * NS
PW = N_PTS // NW
C = 1024
NCHUNK = PW // C

_f32 = jnp.float32
_i32 = jnp.int32


@functools.partial(
    pl.kernel,
    out_type=[
        jax.ShapeDtypeStruct((N_PTS, 3), _f32),
        jax.ShapeDtypeStruct((N_PTS,), _f32),
    ],
    mesh=plsc.VectorSubcoreMesh(core_axis_name="c", subcore_axis_name="s"),
    scratch_types=[
        pltpu.VMEM((C,), _f32),
        pltpu.VMEM((C,), _f32),
        pltpu.VMEM((C,), _f32),
        pltpu.VMEM((C,), _f32),
        pltpu.VMEM((C,), _f32),
        pltpu.VMEM((C,), _f32),
        pltpu.VMEM((C,), _i32),
        pltpu.VMEM((C,), _i32),
        pltpu.VMEM((C,), _i32),
        pltpu.VMEM((8 * C,), _i32),
        pltpu.VMEM((8 * C, 8), _f32),
        pltpu.VMEM((C, 3), _f32),
        pltpu.VMEM((C,), _f32),
        pltpu.SemaphoreType.DMA,
    ],
    compiler_params=pltpu.CompilerParams(
        use_tc_tiling_on_sc=False, needs_layout_passes=False),
)
def _plenoxel_sc(ptx_hbm, pty_hbm, ptz_hbm, cb_hbm, col_hbm, sig_hbm,
                 px, py, pz, wx, wy, wz, mk, pb0, pb1, idx, rows, colb, sigb,
                 sem):
    wid = lax.axis_index("s") * NC + lax.axis_index("c")
    iota = lax.iota(_i32, L)

    @pl.loop(0, NCHUNK)
    def _chunk(ci):
        base = wid * PW + ci * C
        pltpu.sync_copy(ptx_hbm.at[pl.ds(base, C)], px)
        pltpu.sync_copy(pty_hbm.at[pl.ds(base, C)], py)
        pltpu.sync_copy(ptz_hbm.at[pl.ds(base, C)], pz)

        @pl.loop(0, C // L)
        def _prep(v):
            o = v * L
            sl = pl.ds(o, L)
            a = px[sl] * 0.5
            b = py[sl] * 0.5
            c = pz[sl] * 0.5

            def prep_dim(t):
                x = (t + 0.5) * float(RES - 1)
                t0 = jnp.clip(x.astype(_i32), 0, RES - 1)
                t1 = jnp.minimum(t0 + 1, RES - 1)
                w = x - t0.astype(_f32)
                return t0, t1, w

            i0, i1, wxa = prep_dim(a)
            j0, j1, wya = prep_dim(b)
            k0, k1, wza = prep_dim(c)
            f000 = (i0 << 16) + (j0 << 8) + k0
            di = (i1 - i0) << 16
            dj = (j1 - j0) << 8
            dk = k1 - k0
            f001 = f000 + dk
            idx[pl.ds(0 * C + o, L)] = f000 >> 1
            idx[pl.ds(1 * C + o, L)] = f001 >> 1
            idx[pl.ds(2 * C + o, L)] = (f000 + dj) >> 1
            idx[pl.ds(3 * C + o, L)] = (f001 + dj) >> 1
            idx[pl.ds(4 * C + o, L)] = (f000 + di) >> 1
            idx[pl.ds(5 * C + o, L)] = (f001 + di) >> 1
            idx[pl.ds(6 * C + o, L)] = (f000 + di + dj) >> 1
            idx[pl.ds(7 * C + o, L)] = (f001 + di + dj) >> 1
            pb0[sl] = (f000 & 1) << 2
            pb1[sl] = (f001 & 1) << 2
            wx[sl] = wxa
            wy[sl] = wya
            wz[sl] = wza
            one = jnp.full((L,), 1, _i32)
            zro = jnp.zeros((L,), _i32)
            ca = jnp.abs(a) < 0.5
            cb = jnp.abs(b) < 0.5
            cc = jnp.abs(c) < 0.5
            mk[sl] = jnp.where(ca, jnp.where(cb, jnp.where(cc, one, zro), zro),
                               zro)

        pltpu.async_copy(cb_hbm.at[idx], rows, sem).wait()

        @pl.loop(0, C // L)
        def _blend(v):
            o = v * L
            sl = pl.ds(o, L)
            wxa = wx[sl]
            wya = wy[sl]
            wza = wz[sl]
            m = mk[sl] != 0
            cb0v = pb0[sl]
            cb1v = pb1[sl]
            rbase = o + iota

            def corner(cn, colv):
                return plsc.load_gather(rows, [rbase + (cn * C), colv])

            feats = []
            for f in range(4):
                c0f = cb0v + f
                c1f = cb1v + f
                c000 = corner(0, c0f)
                c001 = corner(1, c1f)
                c010 = corner(2, c0f)
                c011 = corner(3, c1f)
                c100 = corner(4, c0f)
                c101 = corner(5, c1f)
                c110 = corner(6, c0f)
                c111 = corner(7, c1f)
                c00 = c000 * (1.0 - wza) + c001 * wza
                c01 = c010 * (1.0 - wza) + c011 * wza
                c10 = c100 * (1.0 - wza) + c101 * wza
                c11 = c110 * (1.0 - wza) + c111 * wza
                c0 = c00 * (1.0 - wya) + c01 * wya
                c1 = c10 * (1.0 - wya) + c11 * wya
                feats.append(c0 * (1.0 - wxa) + c1 * wxa)

            zero = jnp.zeros((L,), _f32)
            for f in range(3):
                plsc.store_scatter(
                    colb, [rbase, jnp.full((L,), f, _i32)],
                    jnp.where(m, feats[f], zero))
            sigb[sl] = jnp.where(m, jnp.exp(feats[3]), zero)

        pltpu.sync_copy(colb, col_hbm.at[pl.ds(base, C)])
        pltpu.sync_copy(sigb, sig_hbm.at[pl.ds(base, C)])


def kernel(pts, d, cb0, cb1, cb2):
    del d, cb0, cb1
    cb8 = cb2.reshape(RES ** 3 // 2, 8)
    col, sig = _plenoxel_sc(pts[:, 0], pts[:, 1], pts[:, 2], cb8)
    return (col, sig[:, None])
